# baseline (speedup 1.0000x reference)
import jax
import jax.numpy as jnp
from jax import lax
from jax.experimental import pallas as pl
from jax.experimental.pallas import tpu as pltpu
from jax.experimental.pallas import tpu_sc as plsc

N = 10000
NP = 10240
E = 320000
D = 128
H1 = 256
EMB = 128
MLP_H = 64
B = 16384
LAM = 0.5

NC = 2
NS = 16
NW = NC * NS
RPT = NP // NW
HC = 256
SUB = 2000
NSUB = E // SUB
K = 64
LISTC = SUB + K

_MESH = plsc.VectorSubcoreMesh(
    core_axis_name="c", subcore_axis_name="s", num_cores=NC, num_subcores=NS
)
_CPARAMS = pltpu.CompilerParams(needs_layout_passes=False)


def _spmm_body(x_hbm, dst_hbm, src_hbm, val_hbm, out_hbm,
               acc, src_list, ldst_list, val_list,
               dsub, ssub, vsub, rows_v, idx_s, sem):
    w = lax.axis_index("c") * NS + lax.axis_index("s")
    lo = w * RPT

    def _zz(r, _):
        for jj in range(HC // 16):
            acc[r, pl.ds(jj * 16, 16)] = jnp.zeros((16,), jnp.float32)
        return 0
    lax.fori_loop(0, RPT, _zz, 0)

    def sub_body(sub, _):
        off = pl.multiple_of(sub * SUB, 8)
        pltpu.sync_copy(dst_hbm.at[pl.ds(off, SUB)], dsub)
        pltpu.sync_copy(src_hbm.at[pl.ds(off, SUB)], ssub)
        pltpu.sync_copy(val_hbm.at[pl.ds(off, SUB)], vsub)

        def scan_g(g, m):
            go = pl.multiple_of(g * 16, 16)
            d16 = dsub[pl.ds(go, 16)]
            s16 = ssub[pl.ds(go, 16)]
            v16 = vsub[pl.ds(go, 16)]
            msk = (d16 >= lo) & (d16 < lo + RPT)
            m32 = msk.astype(jnp.int32)
            pos = jnp.full((16,), m, jnp.int32) + plsc.cumsum(m32) - 1
            plsc.store_scatter(src_list, [pos], s16, mask=msk)
            plsc.store_scatter(ldst_list, [pos], d16 - lo, mask=msk)
            plsc.store_scatter(val_list, [pos], v16, mask=msk)
            return m + jnp.sum(m32)

        m = lax.fori_loop(0, SUB // 16, scan_g, jnp.int32(0))

        zi = jnp.zeros((16,), jnp.int32)
        zf = jnp.zeros((16,), jnp.float32)
        for t in range(K // 16):
            pos = jnp.full((16,), m + t * 16, jnp.int32) + lax.iota(jnp.int32, 16)
            plsc.store_scatter(src_list, [pos], zi)
            plsc.store_scatter(ldst_list, [pos], zi)
            plsc.store_scatter(val_list, [pos], zf)

        nch = (m + (K - 1)) // K

        def chunk(j, _):
            cb = pl.multiple_of(j * K, 16)
            for t in range(K // 16):
                idx_s[pl.ds(t * 16, 16)] = src_list[pl.ds(cb + t * 16, 16)]
            pltpu.async_copy(x_hbm.at[idx_s], rows_v, sem).wait()

            def edge(e, _):
                ldst16 = plsc.load_gather(
                    ldst_list, [jnp.full((16,), cb + e, jnp.int32)])
                val16 = plsc.load_gather(
                    val_list, [jnp.full((16,), cb + e, jnp.int32)])
                for c in range(HC // 16):
                    cols = (jnp.full((16,), c * 16, jnp.int32)
                            + lax.iota(jnp.int32, 16))
                    x16 = rows_v[e, pl.ds(c * 16, 16)] * val16
                    plsc.addupdate_scatter(acc, [ldst16, cols], x16)
                return 0

            lax.fori_loop(0, K, edge, 0)
            return 0

        lax.fori_loop(0, nch, chunk, 0)
        return 0

    lax.fori_loop(0, NSUB, sub_body, 0)

    pltpu.sync_copy(acc, out_hbm.at[pl.ds(lo, RPT)])


_spmm = pl.kernel(
    _spmm_body,
    out_type=jax.ShapeDtypeStruct((NP, HC), jnp.float32),
    mesh=_MESH,
    compiler_params=_CPARAMS,
    scratch_types=[
        pltpu.VMEM((RPT, HC), jnp.float32),
        pltpu.VMEM((LISTC,), jnp.int32),
        pltpu.VMEM((LISTC,), jnp.int32),
        pltpu.VMEM((LISTC,), jnp.float32),
        pltpu.VMEM((SUB,), jnp.int32),
        pltpu.VMEM((SUB,), jnp.int32),
        pltpu.VMEM((SUB,), jnp.float32),
        pltpu.VMEM((K, HC), jnp.float32),
        pltpu.VMEM((K,), jnp.int32),
        pltpu.SemaphoreType.DMA,
    ],
)


_BLK = 1024
_PREC = lax.Precision.HIGHEST


def _stage_a_body(rna_ref, atac_ref, wr_ref, wa_ref, o1_ref, o2_ref):
    o1_ref[...] = jnp.dot(rna_ref[...], wr_ref[...],
                          preferred_element_type=jnp.float32, precision=_PREC)
    o2_ref[...] = jnp.dot(atac_ref[...], wa_ref[...],
                          preferred_element_type=jnp.float32, precision=_PREC)


def _stage_b_body(s1r_ref, s1a_ref, wr_ref, wa_ref, o_ref):
    a = jnp.maximum(s1r_ref[...], 0.0)
    b = jnp.maximum(s1a_ref[...], 0.0)
    o_ref[:, :EMB] = jnp.dot(a, wr_ref[...],
                             preferred_element_type=jnp.float32,
                             precision=_PREC)
    o_ref[:, EMB:] = jnp.dot(b, wa_ref[...],
                             preferred_element_type=jnp.float32,
                             precision=_PREC)


def _stage_c_body(s2_ref, w1a_ref, w1b_ref, t_ref):
    a = jnp.maximum(s2_ref[:, :EMB], 0.0)
    b = jnp.maximum(s2_ref[:, EMB:], 0.0)
    h = (1.0 - LAM) * a + LAM * b
    t_ref[:, :MLP_H] = jnp.dot(h, w1a_ref[...],
                               preferred_element_type=jnp.float32,
                               precision=_PREC)
    t_ref[:, MLP_H:] = jnp.dot(h, w1b_ref[...],
                               preferred_element_type=jnp.float32,
                               precision=_PREC)


def _stage_a(rna, atac, wr, wa):
    return pl.pallas_call(
        _stage_a_body,
        grid=(NP // _BLK,),
        in_specs=[
            pl.BlockSpec((_BLK, D), lambda i: (i, 0)),
            pl.BlockSpec((_BLK, D), lambda i: (i, 0)),
            pl.BlockSpec((D, H1), lambda i: (0, 0)),
            pl.BlockSpec((D, H1), lambda i: (0, 0)),
        ],
        out_specs=[
            pl.BlockSpec((_BLK, H1), lambda i: (i, 0)),
            pl.BlockSpec((_BLK, H1), lambda i: (i, 0)),
        ],
        out_shape=[
            jax.ShapeDtypeStruct((NP, H1), jnp.float32),
            jax.ShapeDtypeStruct((NP, H1), jnp.float32),
        ],
    )(rna, atac, wr, wa)


def _stage_b(s1r, s1a, wr2, wa2):
    return pl.pallas_call(
        _stage_b_body,
        grid=(NP // _BLK,),
        in_specs=[
            pl.BlockSpec((_BLK, H1), lambda i: (i, 0)),
            pl.BlockSpec((_BLK, H1), lambda i: (i, 0)),
            pl.BlockSpec((H1, EMB), lambda i: (0, 0)),
            pl.BlockSpec((H1, EMB), lambda i: (0, 0)),
        ],
        out_specs=pl.BlockSpec((_BLK, 2 * EMB), lambda i: (i, 0)),
        out_shape=jax.ShapeDtypeStruct((NP, 2 * EMB), jnp.float32),
    )(s1r, s1a, wr2, wa2)


def _stage_c(s2, w1a, w1b):
    return pl.pallas_call(
        _stage_c_body,
        grid=(NP // _BLK,),
        in_specs=[
            pl.BlockSpec((_BLK, 2 * EMB), lambda i: (i, 0)),
            pl.BlockSpec((EMB, MLP_H), lambda i: (0, 0)),
            pl.BlockSpec((EMB, MLP_H), lambda i: (0, 0)),
        ],
        out_specs=pl.BlockSpec((_BLK, 2 * MLP_H), lambda i: (i, 0)),
        out_shape=jax.ShapeDtypeStruct((NP, 2 * MLP_H), jnp.float32),
    )(s2, w1a, w1b)


PPW = B // NW
PH = PPW // 2


def _pairs_body(t_hbm, s0_hbm, s1_hbm, b1_hbm, w2_hbm, out_hbm,
                e1b, e2b, i0, i1, b1v, w2v, ob, sem):
    wid = lax.axis_index("c") * NS + lax.axis_index("s")
    base = wid * PPW
    pltpu.sync_copy(b1_hbm, b1v)
    pltpu.sync_copy(w2_hbm, w2v)

    for half in range(2):
        hb = base + half * PH
        pltpu.sync_copy(s0_hbm.at[pl.ds(hb, PH)], i0)
        pltpu.sync_copy(s1_hbm.at[pl.ds(hb, PH)], i1)
        pltpu.async_copy(t_hbm.at[i0], e1b, sem).wait()
        pltpu.async_copy(t_hbm.at[i1], e2b, sem).wait()

        def pair(p, _):
            acc = jnp.zeros((16,), jnp.float32)
            for c in range(MLP_H // 16):
                va = e1b[p, pl.ds(c * 16, 16)]
                vb = e2b[p, pl.ds(MLP_H + c * 16, 16)]
                bb = b1v[pl.ds(c * 16, 16)]
                ww = w2v[pl.ds(c * 16, 16)]
                tt = jnp.maximum(va + vb + bb, 0.0)
                acc = acc + tt * ww
            ssum = jnp.maximum(jnp.sum(acc), 0.0)
            plsc.store_scatter(
                ob, [jnp.full((16,), half * PH + p, jnp.int32)],
                jnp.full((16,), ssum, jnp.float32),
                mask=lax.iota(jnp.int32, 16) == 0)
            return 0

        lax.fori_loop(0, PH, pair, 0)

    sync_out = out_hbm.at[pl.ds(base, PPW)]
    pltpu.sync_copy(ob, sync_out)


_pairs = pl.kernel(
    _pairs_body,
    out_type=jax.ShapeDtypeStruct((B,), jnp.float32),
    mesh=_MESH,
    compiler_params=_CPARAMS,
    scratch_types=[
        pltpu.VMEM((PH, 2 * MLP_H), jnp.float32),
        pltpu.VMEM((PH, 2 * MLP_H), jnp.float32),
        pltpu.VMEM((PH,), jnp.int32),
        pltpu.VMEM((PH,), jnp.int32),
        pltpu.VMEM((MLP_H,), jnp.float32),
        pltpu.VMEM((MLP_H,), jnp.float32),
        pltpu.VMEM((PPW,), jnp.float32),
        pltpu.SemaphoreType.DMA,
    ],
)


def kernel(edge_index, adj_vals, train_sample, rna, atac,
           W_rna1, W_rna2, W_atac1, W_atac2, mlp_w1, mlp_b1, mlp_w2):
    dst = edge_index[0]
    src = edge_index[1]
    rna_p = jnp.pad(rna, ((0, NP - N), (0, 0)))
    atac_p = jnp.pad(atac, ((0, NP - N), (0, 0)))

    p1r, p1a = _stage_a(rna_p, atac_p, W_rna1, W_atac1)
    s1r = _spmm(p1r, dst, src, adj_vals)
    s1a = _spmm(p1a, dst, src, adj_vals)
    p2 = _stage_b(s1r, s1a, W_rna2, W_atac2)
    s2 = _spmm(p2, dst, src, adj_vals)
    t = _stage_c(s2, mlp_w1[:EMB], mlp_w1[EMB:])

    out = _pairs(t, train_sample[:, 0], train_sample[:, 1],
                 mlp_b1, mlp_w2[:, 0])
    return out.reshape(B, 1)
